# Initial kernel scaffold; baseline (speedup 1.0000x reference)
import jax
import jax.numpy as jnp
from jax.experimental import pallas as pl


def kernel(x, edge_attr, W_emb, b_emb, Wf, bf, Ws, bs, cg_g, cg_b, cg_m, cg_v, bn_g, bn_b, bn_m, bn_v, W_pool, b_pool, Wfh1, bfh1, Wfh2, bfh2, Wfh3, bfh3, Wdh1, bdh1, Wdh2, bdh2, edge_index, batch):
    raise NotImplementedError("write your pallas kernel here")


# speedup vs baseline: 1.7380x; 1.7380x over previous
import functools

import jax
import jax.numpy as jnp
from jax import lax
from jax.experimental import pallas as pl
from jax.experimental.pallas import tpu as pltpu
from jax.experimental.pallas import tpu_sc as plsc

_HI = jax.lax.Precision.HIGHEST
_LO = None

N = 10000
E = 320000
B = 128
D = 128
NC = 6

_NW = 32
_EPW = E // _NW
_CH = 80
_NCHUNK = _EPW // _CH
_ROWCH = 200
_NROWCH = N // _ROWCH

_mesh = plsc.VectorSubcoreMesh(core_axis_name="c", subcore_axis_name="s")


@functools.partial(
    pl.kernel,
    out_type=(
        jax.ShapeDtypeStruct((E, 2 * D), jnp.float32),
        jax.ShapeDtypeStruct((E, 2 * D), jnp.float32),
    ),
    mesh=_mesh,
    scratch_types=[
        pltpu.VMEM((_CH,), jnp.int32),
        pltpu.VMEM((_CH,), jnp.int32),
        pltpu.VMEM((_CH, 2 * D), jnp.float32),
        pltpu.VMEM((_CH, 2 * D), jnp.float32),
        pltpu.SemaphoreType.DMA,
        pltpu.SemaphoreType.DMA,
    ],
)
def _sc_gather(pd_hbm, ps_hbm, dst_hbm, src_hbm, gd_hbm, gs_hbm,
               idxd_v, idxs_v, bufd_v, bufs_v, semd, sems):
    wid = lax.axis_index("s") * 2 + lax.axis_index("c")
    base = wid * _EPW

    def chunk(j, carry):
        off = base + j * _CH
        pltpu.sync_copy(dst_hbm.at[pl.ds(off, _CH)], idxd_v)
        pltpu.sync_copy(src_hbm.at[pl.ds(off, _CH)], idxs_v)
        cpd = pltpu.async_copy(pd_hbm.at[idxd_v], bufd_v, semd)
        cps = pltpu.async_copy(ps_hbm.at[idxs_v], bufs_v, sems)
        cpd.wait()
        cps.wait()
        pltpu.sync_copy(bufd_v, gd_hbm.at[pl.ds(off, _CH)])
        pltpu.sync_copy(bufs_v, gs_hbm.at[pl.ds(off, _CH)])
        return carry

    lax.fori_loop(0, _NCHUNK, chunk, 0)


@functools.partial(
    pl.kernel,
    out_type=jax.ShapeDtypeStruct((2, N, D), jnp.float32),
    mesh=_mesh,
    scratch_types=[
        pltpu.VMEM_SHARED((N, D), jnp.float32),
        pltpu.VMEM((_CH,), jnp.int32),
        pltpu.VMEM((_CH, D), jnp.float32),
    ],
)
def _sc_scatter(msg_hbm, dst_hbm, zeros_hbm, out_hbm, acc_sh, idx_v, rows_v):
    cid = lax.axis_index("c")
    sid = lax.axis_index("s")
    wid = sid * 2 + cid
    base = wid * _EPW

    for r in range(_NROWCH):
        @pl.when(sid == (r % 16))
        def _():
            pltpu.sync_copy(zeros_hbm.at[pl.ds(r * _ROWCH, _ROWCH)],
                            acc_sh.at[pl.ds(r * _ROWCH, _ROWCH)])
    plsc.subcore_barrier()

    def chunk(j, carry):
        off = base + j * _CH
        pltpu.sync_copy(dst_hbm.at[pl.ds(off, _CH)], idx_v)
        pltpu.sync_copy(msg_hbm.at[pl.ds(off, _CH)], rows_v)
        pltpu.sync_copy(rows_v, acc_sh.at[idx_v], add=True)
        return carry

    lax.fori_loop(0, _NCHUNK, chunk, 0)
    plsc.subcore_barrier()

    for r in range(_NROWCH):
        @pl.when(sid == (r % 16))
        def _():
            pltpu.sync_copy(acc_sh.at[pl.ds(r * _ROWCH, _ROWCH)],
                            out_hbm.at[cid].at[pl.ds(r * _ROWCH, _ROWCH)])


def _mm_bias_body(a_ref, w_ref, b_ref, o_ref):
    o_ref[...] = (jnp.dot(a_ref[...], w_ref[...],
                          preferred_element_type=jnp.float32,
                          precision=_LO) + b_ref[...])


def _mm_bias(a, w, b2d, bm):
    m, k = a.shape
    n = w.shape[1]
    return pl.pallas_call(
        _mm_bias_body,
        grid=(m // bm,),
        in_specs=[
            pl.BlockSpec((bm, k), lambda i: (i, 0)),
            pl.BlockSpec((k, n), lambda i: (0, 0)),
            pl.BlockSpec((1, n), lambda i: (0, 0)),
        ],
        out_specs=pl.BlockSpec((bm, n), lambda i: (i, 0)),
        out_shape=jax.ShapeDtypeStruct((m, n), jnp.float32),
    )(a, w, b2d)


def _proj_body(h_ref, w_ref, o_ref):
    o_ref[0] = jnp.dot(h_ref[...], w_ref[...],
                       preferred_element_type=jnp.float32, precision=_LO)


def _proj(h, wn, bm):
    return pl.pallas_call(
        _proj_body,
        grid=(N // bm, 2),
        in_specs=[
            pl.BlockSpec((bm, D), lambda i, j: (i, 0)),
            pl.BlockSpec((D, 2 * D), lambda i, j: (0, j)),
        ],
        out_specs=pl.BlockSpec((1, bm, 2 * D), lambda i, j: (j, i, 0)),
        out_shape=jax.ShapeDtypeStruct((2, N, 2 * D), jnp.float32),
    )(h, wn)


def _msg_body(ea_ref, we_ref, b2_ref, gd_ref, gs_ref, o_ref):
    z = (jnp.dot(ea_ref[...], we_ref[...], preferred_element_type=jnp.float32,
                 precision=_LO)
         + gd_ref[...] + gs_ref[...] + b2_ref[...])
    o_ref[...] = jax.nn.sigmoid(z[:, :D]) * jax.nn.softplus(z[:, D:])


def _msg(ea, we, b2, gd, gs, be):
    ke = ea.shape[1]
    return pl.pallas_call(
        _msg_body,
        grid=(E // be,),
        in_specs=[
            pl.BlockSpec((be, ke), lambda i: (i, 0)),
            pl.BlockSpec((ke, 2 * D), lambda i: (0, 0)),
            pl.BlockSpec((1, 2 * D), lambda i: (0, 0)),
            pl.BlockSpec((be, 2 * D), lambda i: (i, 0)),
            pl.BlockSpec((be, 2 * D), lambda i: (i, 0)),
        ],
        out_specs=pl.BlockSpec((be, D), lambda i: (i, 0)),
        out_shape=jax.ShapeDtypeStruct((E, D), jnp.float32),
    )(ea, we, b2, gd, gs)


def _upd_body(with_res, p_ref, h_ref, sc_ref, o_ref):
    a = (p_ref[0] + p_ref[1]) * sc_ref[0:1, :] + sc_ref[1:2, :] + h_ref[...]
    a = a * sc_ref[2:3, :] + sc_ref[3:4, :]
    if with_res:
        a = a + h_ref[...]
    o_ref[...] = a


def _upd(partials, h, sc, with_res, bm):
    return pl.pallas_call(
        functools.partial(_upd_body, with_res),
        grid=(N // bm,),
        in_specs=[
            pl.BlockSpec((2, bm, D), lambda i: (0, i, 0)),
            pl.BlockSpec((bm, D), lambda i: (i, 0)),
            pl.BlockSpec((4, D), lambda i: (0, 0)),
        ],
        out_specs=pl.BlockSpec((bm, D), lambda i: (i, 0)),
        out_shape=jax.ShapeDtypeStruct((N, D), jnp.float32),
    )(partials, h, sc)


_PB = 1000


def _pool_body(h_ref, b_ref, o_ref, s_ref, m_ref, c_ref):
    step = pl.program_id(0)

    @pl.when(step == 0)
    def _():
        s_ref[...] = jnp.zeros_like(s_ref)
        m_ref[...] = jnp.full_like(m_ref, -1e30)
        c_ref[...] = jnp.zeros_like(c_ref)

    bvec = b_ref[0, 0, :]
    ids = lax.broadcasted_iota(jnp.int32, (B, _PB), 0)
    mf = (bvec[None, :] == ids).astype(jnp.float32)
    c_ref[...] += jnp.sum(mf, axis=1, keepdims=True)
    s_ref[...] += jnp.dot(mf, h_ref[...], preferred_element_type=jnp.float32,
                          precision=_HI)
    h = h_ref[...]
    penal = (mf - 1.0) * 1e30
    for g in range(B // 8):
        cand = jnp.max(penal[8 * g:8 * g + 8][:, :, None] + h[None, :, :],
                       axis=1)
        m_ref[8 * g:8 * g + 8] = jnp.maximum(m_ref[8 * g:8 * g + 8], cand)

    @pl.when(step == pl.num_programs(0) - 1)
    def _():
        cnt = c_ref[...]
        mean = s_ref[...] / jnp.maximum(cnt, 1.0)
        mx = jnp.where(cnt > 0, m_ref[...], 0.0)
        o_ref[...] = jnp.concatenate([mean, mx], axis=1)


def _pool(h, batch3):
    return pl.pallas_call(
        _pool_body,
        grid=(N // _PB,),
        in_specs=[
            pl.BlockSpec((_PB, D), lambda i: (i, 0)),
            pl.BlockSpec((1, 1, _PB), lambda i: (i, 0, 0)),
        ],
        out_specs=pl.BlockSpec((B, 2 * D), lambda i: (0, 0)),
        out_shape=jax.ShapeDtypeStruct((B, 2 * D), jnp.float32),
        scratch_shapes=[
            pltpu.VMEM((B, D), jnp.float32),
            pltpu.VMEM((B, D), jnp.float32),
            pltpu.VMEM((B, 1), jnp.float32),
        ],
    )(h, batch3)


def _heads_body(p_ref, wp, bp, w1, b1, w2, b2, w3, b3, wd1, bd1, wd2, bd2,
                of_ref, od_ref):
    dot = functools.partial(jnp.dot, preferred_element_type=jnp.float32,
                            precision=_LO)
    emb = jax.nn.relu(dot(p_ref[...], wp[...]) + bp[...])
    f = jax.nn.silu(dot(emb, w1[...]) + b1[...])
    f = jax.nn.silu(dot(f, w2[...]) + b2[...])
    of_ref[...] = dot(f, w3[...]) + b3[...]
    d = jax.nn.silu(dot(emb, wd1[...]) + bd1[...])
    od_ref[...] = dot(d, wd2[...]) + bd2[...]


def _heads(pooled, args):
    return pl.pallas_call(
        _heads_body,
        out_shape=(
            jax.ShapeDtypeStruct((B, 8), jnp.float32),
            jax.ShapeDtypeStruct((B, 384), jnp.float32),
        ),
    )(pooled, *args)


def kernel(x, edge_attr, W_emb, b_emb, Wf, bf, Ws, bs, cg_g, cg_b, cg_m, cg_v,
           bn_g, bn_b, bn_m, bn_v, W_pool, b_pool, Wfh1, bfh1, Wfh2, bfh2,
           Wfh3, bfh3, Wdh1, bdh1, Wdh2, bdh2, edge_index, batch):
    f32 = jnp.float32
    src = edge_index[0]
    dst = edge_index[1]

    xp = jnp.pad(x, ((0, 0), (0, 128 - x.shape[1])))
    wembp = jnp.pad(W_emb, ((0, 128 - W_emb.shape[0]), (0, 0)))
    eap = jnp.pad(edge_attr, ((0, 0), (0, 48 - edge_attr.shape[1])))
    zeros_nd = jnp.zeros((N, D), f32)

    h = _mm_bias(xp, wembp, b_emb[None, :], 2000)

    for i in range(NC):
        wf, ws = Wf[i], Ws[i]
        wn = jnp.concatenate(
            [wf[0:D], ws[0:D], wf[D:2 * D], ws[D:2 * D]], axis=1)
        we = jnp.pad(jnp.concatenate([wf[2 * D:], ws[2 * D:]], axis=1),
                     ((0, 7), (0, 0)))
        b2 = jnp.concatenate([bf[i], bs[i]])[None, :]
        s1 = cg_g[i] * lax.rsqrt(cg_v[i] + 1e-5)
        t1 = cg_b[i] - cg_m[i] * s1
        s2 = bn_g[i] * lax.rsqrt(bn_v[i] + 1e-5)
        t2 = bn_b[i] - bn_m[i] * s2
        sc = jnp.stack([s1, t1, s2, t2])

        p3 = _proj(h, wn, 2000)
        gd, gs = _sc_gather(p3[0], p3[1], dst, src)
        msg = _msg(eap, we, b2, gd, gs, 2000)
        partials = _sc_scatter(msg, dst, zeros_nd)
        h = _upd(partials, h, sc, i > 0, 2000)

    batch3 = batch.reshape(N // _PB, 1, _PB)
    pooled = _pool(h, batch3)

    head_args = (W_pool, b_pool[None, :], Wfh1, bfh1[None, :], Wfh2,
                 bfh2[None, :], jnp.pad(Wfh3, ((0, 0), (0, 7))),
                 jnp.pad(bfh3, (0, 7))[None, :], Wdh1, bdh1[None, :],
                 Wdh2, bdh2[None, :])
    form8, dos = _heads(pooled, head_args)
    return form8[:, :1], dos.reshape(B, 3, D)
